# Initial kernel scaffold; baseline (speedup 1.0000x reference)
import jax
import jax.numpy as jnp
from jax.experimental import pallas as pl


def kernel(x, mask, A_in_events, A_in_clusters, A_src_in_product, A_src_in_sta, locs_cart, srcs_cart, params):
    raise NotImplementedError("write your pallas kernel here")


# speedup vs baseline: 1.0190x; 1.0190x over previous
import functools

import jax
import jax.numpy as jnp
from jax.experimental import pallas as pl
from jax.experimental.pallas import tpu as pltpu


def _cdiv(a, b):
    return (a + b - 1) // b


@functools.partial(jax.jit, static_argnames=("apply_act",))
def _pl_linear(x, W, b, a, apply_act=True):
    N, K = x.shape
    M = W.shape[1]
    R = 2048
    grid = _cdiv(N, R)

    def body(x_ref, w_ref, b_ref, a_ref, o_ref):
        y = jnp.dot(x_ref[...], w_ref[...], preferred_element_type=jnp.float32)
        y = y + b_ref[...]
        if apply_act:
            av = a_ref[0, 0]
            y = jnp.where(y >= 0, y, av * y)
        o_ref[...] = y

    return pl.pallas_call(
        body,
        grid=(grid,),
        in_specs=[
            pl.BlockSpec((R, K), lambda i: (i, 0)),
            pl.BlockSpec((K, M), lambda i: (0, 0)),
            pl.BlockSpec((1, M), lambda i: (0, 0)),
            pl.BlockSpec((1, 1), lambda i: (0, 0)),
        ],
        out_specs=pl.BlockSpec((R, M), lambda i: (i, 0)),
        out_shape=jax.ShapeDtypeStruct((N, M), jnp.float32),
    )(x, W, b.reshape(1, M), jnp.asarray(a, jnp.float32).reshape(1, 1))


def _linear(p, x, a=None):
    if a is None:
        return _pl_linear(x, p['W'], p['b'], jnp.float32(0.0), apply_act=False)
    return _pl_linear(x, p['W'], p['b'], a, apply_act=True)


def _prelu(x, a):
    return jnp.where(x >= 0, x, a * x)


def _scatter_mean(msgs, dst, num):
    s = jax.ops.segment_sum(msgs, dst, num_segments=num)
    c = jax.ops.segment_sum(jnp.ones((msgs.shape[0],), msgs.dtype), dst, num_segments=num)
    return s / jnp.maximum(c, 1.0)[:, None]


def _data_agg(p, x, mask, A_sta, A_src, pr_sta, pr_src, n):
    tr = jnp.concatenate([x, mask], axis=1)
    tr = _linear(p['init_trns'], tr, p['a_init'])

    def prop(edges, xin, ea):
        msg = _linear(p['merge'], jnp.concatenate([xin[edges[0]], ea], axis=1), p['a_merge'])
        return _scatter_mean(msg, edges[1], n)

    tr1 = _linear(p['l1_t1_2'], jnp.concatenate([tr, prop(A_sta, _prelu(tr, p['a11']), pr_sta), mask], axis=1))
    tr2 = _linear(p['l1_t2_2'], jnp.concatenate([tr, prop(A_src, _prelu(tr, p['a12']), pr_src), mask], axis=1))
    tr = _prelu(jnp.concatenate([tr1, tr2], axis=1), p['a1'])
    tr1 = _linear(p['l2_t1_2'], jnp.concatenate([tr, prop(A_sta, _linear(p['l2_t1_1'], tr, p['a21']), pr_sta), mask], axis=1))
    tr2 = _linear(p['l2_t2_2'], jnp.concatenate([tr, prop(A_src, _linear(p['l2_t2_1'], tr, p['a22']), pr_src), mask], axis=1))
    return _prelu(jnp.concatenate([tr1, tr2], axis=1), p['a2'])


def kernel(x, mask, A_in_events, A_in_clusters, A_src_in_product, A_src_in_sta, locs_cart, srcs_cart, params):
    n = x.shape[0]
    rel = locs_cart[A_src_in_sta[0]] - srcs_cart[A_src_in_sta[1]]
    x = jnp.concatenate([x, rel], axis=1)
    mask = jnp.concatenate([mask, rel], axis=1)
    e = params['embed']
    mask = _linear(e['l2'], _linear(e['l1'], mask, e['a']))
    sta_of_node = A_src_in_sta[0]
    src_of_node = A_src_in_sta[1]
    pr_sta = locs_cart[sta_of_node[A_in_events[0]]] - locs_cart[sta_of_node[A_in_events[1]]]
    pr_src = srcs_cart[src_of_node[A_in_clusters[0]]] - srcs_cart[src_of_node[A_in_clusters[1]]]
    for p in params['aggs']:
        x = _data_agg(p, x, mask, A_in_events, A_in_clusters, pr_sta, pr_src, n)
    b = params['bip']
    xcat = jnp.concatenate([x, mask], axis=1)
    pos_j_all = locs_cart[A_src_in_sta[0]]
    ej = A_src_in_product[0]
    ei = A_src_in_product[1]
    h = jnp.concatenate([xcat[ej], srcs_cart[ei] - pos_j_all[ej]], axis=1)
    h = _linear(b['fc1b'], _linear(b['fc1a'], h, b['a_fc1']))
    h = _prelu(h, b['a1'])
    agg = jax.ops.segment_sum(h, ei, num_segments=srcs_cart.shape[0])
    out = _linear(b['fc2'], agg, b['a2'])
    pr = params['proj']
    return _linear(pr['l2'], _linear(pr['l1'], out, pr['a']))


# speedup vs baseline: 1.7847x; 1.7515x over previous
import functools

import jax
import jax.numpy as jnp
from jax import lax
from jax.experimental import pallas as pl
from jax.experimental.pallas import tpu as pltpu
from jax.experimental.pallas import tpu_sc as plsc

N_NODES = 100000
ACC_ROWS = 100352
JUNK_ROW = 100000
EB_BLK = 128
NSUB = 16


def _cdiv(a, b):
    return (a + b - 1) // b


@functools.partial(jax.jit, static_argnames=("pre_act", "post_act", "split16"))
def _pl_linear(x, W, b, a_pre, a_post, pre_act=False, post_act=False, split16=False):
    N, K = x.shape
    M = W.shape[1]
    R = 2048
    grid = _cdiv(N, R)

    def body(x_ref, w_ref, b_ref, ap_ref, aq_ref, *out_refs):
        xv = x_ref[...]
        if pre_act:
            ap = ap_ref[0, 0]
            xv = jnp.where(xv >= 0, xv, ap * xv)
        y = jnp.dot(xv, w_ref[...], preferred_element_type=jnp.float32)
        y = y + b_ref[...]
        if post_act:
            aq = aq_ref[0, 0]
            y = jnp.where(y >= 0, y, aq * y)
        if split16:
            out_refs[0][...] = y[:, :16]
            out_refs[1][...] = y[:, 16:32]
        else:
            out_refs[0][...] = y

    if split16:
        out_specs = [pl.BlockSpec((R, 16), lambda i: (i, 0)),
                     pl.BlockSpec((R, 16), lambda i: (i, 0))]
        out_shape = [jax.ShapeDtypeStruct((N, 16), jnp.float32),
                     jax.ShapeDtypeStruct((N, 16), jnp.float32)]
    else:
        out_specs = pl.BlockSpec((R, M), lambda i: (i, 0))
        out_shape = jax.ShapeDtypeStruct((N, M), jnp.float32)

    return pl.pallas_call(
        body,
        grid=(grid,),
        in_specs=[
            pl.BlockSpec((R, K), lambda i: (i, 0)),
            pl.BlockSpec((K, M), lambda i: (0, 0)),
            pl.BlockSpec((1, M), lambda i: (0, 0)),
            pl.BlockSpec((1, 1), lambda i: (0, 0)),
            pl.BlockSpec((1, 1), lambda i: (0, 0)),
        ],
        out_specs=out_specs,
        out_shape=out_shape,
    )(x, W, b.reshape(1, M),
      jnp.asarray(a_pre, jnp.float32).reshape(1, 1),
      jnp.asarray(a_post, jnp.float32).reshape(1, 1))


def _linear(p, x, a=None):
    if a is None:
        return _pl_linear(x, p['W'], p['b'], 0.0, 0.0)
    return _pl_linear(x, p['W'], p['b'], 0.0, a, post_act=True)


def _prelu(x, a):
    return jnp.where(x >= 0, x, a * x)


def _make_sc_prop(E_pad, T):
    EB = E_pad // NSUB
    NBLK = EB // EB_BLK
    Z = ACC_ROWS // NSUB

    mesh = plsc.VectorSubcoreMesh(core_axis_name="c", subcore_axis_name="s")

    @functools.partial(
        pl.kernel,
        out_type=jax.ShapeDtypeStruct((2, ACC_ROWS, 16), jnp.float32),
        mesh=mesh,
        compiler_params=pltpu.CompilerParams(use_tc_tiling_on_sc=False),
        scratch_types=[
            pltpu.VMEM((T, 16), jnp.float32),
            pltpu.VMEM((EB_BLK, 16), jnp.float32),
            pltpu.VMEM((EB_BLK, 16), jnp.float32),
            pltpu.VMEM((EB_BLK,), jnp.int32),
            pltpu.VMEM((EB_BLK,), jnp.int32),
            pltpu.VMEM((EB_BLK,), jnp.int32),
            pltpu.VMEM((EB_BLK,), jnp.int32),
            pltpu.VMEM((16,), jnp.float32),
            pltpu.VMEM_SHARED((ACC_ROWS, 16), jnp.float32),
            pltpu.SemaphoreType.DMA,
        ],
    )
    def sc_prop(u0, u1, g0, g1, src_h, dst_h, ts_h, td_h, a_h, zeros_h, out,
                gtab_v, urows_v, msgs_v, si_v, di_v, ts_v, td_v, a_v, acc, sem):
        c = lax.axis_index("c")
        s = lax.axis_index("s")

        pltpu.sync_copy(zeros_h.at[pl.ds(s * Z, Z)], acc.at[pl.ds(s * Z, Z)])
        pltpu.sync_copy(a_h, a_v)

        @pl.when(c == 0)
        def _():
            pltpu.sync_copy(g0, gtab_v)

        @pl.when(c == 1)
        def _():
            pltpu.sync_copy(g1, gtab_v)

        plsc.subcore_barrier()

        def blk(j, carry):
            base = s * EB + j * EB_BLK
            pltpu.sync_copy(src_h.at[pl.ds(base, EB_BLK)], si_v)
            pltpu.sync_copy(dst_h.at[pl.ds(base, EB_BLK)], di_v)
            pltpu.sync_copy(ts_h.at[pl.ds(base, EB_BLK)], ts_v)
            pltpu.sync_copy(td_h.at[pl.ds(base, EB_BLK)], td_v)

            @pl.when(c == 0)
            def _():
                pltpu.async_copy(u0.at[si_v], urows_v, sem).wait()

            @pl.when(c == 1)
            def _():
                pltpu.async_copy(u1.at[si_v], urows_v, sem).wait()

            av = a_v[...]

            def inner(g, carry2):
                ts16 = ts_v[pl.ds(g * 16, 16)]
                td16 = td_v[pl.ds(g * 16, 16)]
                for i in range(16):
                    tsi = ts16[i]
                    tdi = td16[i]
                    z = urows_v[g * 16 + i] + gtab_v[tsi] - gtab_v[tdi]
                    msgs_v[g * 16 + i] = (jnp.maximum(z, 0.0)
                                          + av * jnp.minimum(z, 0.0))
                return carry2

            lax.fori_loop(0, EB_BLK // 16, inner, 0)
            pltpu.sync_copy(msgs_v, acc.at[di_v], add=True)
            return carry

        lax.fori_loop(0, NBLK, blk, 0)
        plsc.subcore_barrier()

        @pl.when(c == 0)
        def _():
            pltpu.sync_copy(acc.at[pl.ds(s * Z, Z)], out.at[0, pl.ds(s * Z, Z)])

        @pl.when(c == 1)
        def _():
            pltpu.sync_copy(acc.at[pl.ds(s * Z, Z)], out.at[1, pl.ds(s * Z, Z)])

    return sc_prop


def _make_sc_count(E_pad):
    EB = E_pad // NSUB
    NBLK = EB // EB_BLK
    Z = ACC_ROWS // NSUB
    mesh = plsc.VectorSubcoreMesh(core_axis_name="c", subcore_axis_name="s")

    @functools.partial(
        pl.kernel,
        out_type=jax.ShapeDtypeStruct((2, ACC_ROWS, 16), jnp.float32),
        mesh=mesh,
        compiler_params=pltpu.CompilerParams(use_tc_tiling_on_sc=False),
        scratch_types=[
            pltpu.VMEM((EB_BLK, 16), jnp.float32),
            pltpu.VMEM((EB_BLK,), jnp.int32),
            pltpu.VMEM_SHARED((ACC_ROWS, 16), jnp.float32),
        ],
    )
    def sc_count(dst_h, ones_h, zeros_h, out, ones_v, di_v, acc):
        c = lax.axis_index("c")
        s = lax.axis_index("s")
        pltpu.sync_copy(zeros_h.at[pl.ds(s * Z, Z)], acc.at[pl.ds(s * Z, Z)])
        pltpu.sync_copy(ones_h, ones_v)
        plsc.subcore_barrier()

        def blk(j, carry):
            base = s * EB + j * EB_BLK
            pltpu.sync_copy(dst_h.at[pl.ds(base, EB_BLK)], di_v)
            pltpu.sync_copy(ones_v, acc.at[di_v], add=True)
            return carry

        lax.fori_loop(0, NBLK, blk, 0)
        plsc.subcore_barrier()

        @pl.when(c == 0)
        def _():
            pltpu.sync_copy(acc.at[pl.ds(s * Z, Z)], out.at[0, pl.ds(s * Z, Z)])

        @pl.when(c == 1)
        def _():
            pltpu.sync_copy(acc.at[pl.ds(s * Z, Z)], out.at[1, pl.ds(s * Z, Z)])

    return sc_count


def _pad_edges(v, E_pad, fill):
    E = v.shape[0]
    return jnp.pad(v, (0, E_pad - E), constant_values=fill)


def kernel(x, mask, A_in_events, A_in_clusters, A_src_in_product, A_src_in_sta,
           locs_cart, srcs_cart, params):
    n = x.shape[0]
    rel = locs_cart[A_src_in_sta[0]] - srcs_cart[A_src_in_sta[1]]
    x = jnp.concatenate([x, rel], axis=1)
    mask = jnp.concatenate([mask, rel], axis=1)
    e = params['embed']
    mask = _linear(e['l2'], _linear(e['l1'], mask, e['a']))
    sta_of_node = A_src_in_sta[0].astype(jnp.int32)
    src_of_node = A_src_in_sta[1].astype(jnp.int32)

    E = A_in_events.shape[1]
    E_pad = _cdiv(E, NSUB * EB_BLK) * NSUB * EB_BLK
    sc_prop = _make_sc_prop(E_pad, 1024)
    sc_count = _make_sc_count(E_pad)

    zeros_h = jnp.zeros((ACC_ROWS, 16), jnp.float32)
    ones_h = jnp.ones((EB_BLK, 16), jnp.float32)

    def prep_edges(A, tbl_of_node):
        src = _pad_edges(A[0].astype(jnp.int32), E_pad, 0)
        dst = _pad_edges(A[1].astype(jnp.int32), E_pad, JUNK_ROW)
        ts = _pad_edges(tbl_of_node[A[0]], E_pad, 0)
        td = _pad_edges(tbl_of_node[A[1]], E_pad, 0)
        cnt = sc_count(dst, ones_h, zeros_h)[0][:N_NODES, :1]
        recip = 1.0 / jnp.maximum(cnt, 1.0)
        return (src, dst, ts, td), recip

    ev_edges, ev_recip = prep_edges(A_in_events, sta_of_node)
    cl_edges, cl_recip = prep_edges(A_in_clusters, src_of_node)

    locs_pad = jnp.pad(locs_cart, ((0, 1024 - locs_cart.shape[0]), (0, 0)))
    srcs_pad = jnp.pad(srcs_cart, ((0, 1024 - srcs_cart.shape[0]), (0, 0)))

    def prop(p, edges, recip, base_tab, xin_pre, a_in):
        Wm, bm = p['merge']['W'], p['merge']['b']
        W_node = jnp.pad(Wm[:30], ((0, 0), (0, 2)))
        b_node = jnp.pad(bm, (0, 2))
        u0, u1 = _pl_linear(xin_pre, W_node, b_node, a_in, 0.0,
                            pre_act=True, split16=True)
        gt = base_tab @ jnp.pad(Wm[30:34], ((0, 0), (0, 2)))
        gt0, gt1 = gt[:, :16], gt[:, 16:32]
        src, dst, ts, td = edges
        a_vec = jnp.full((16,), p['a_merge'], jnp.float32)
        out = sc_prop(u0, u1, gt0, gt1, src, dst, ts, td, a_vec, zeros_h)
        o = jnp.concatenate([out[0][:N_NODES], out[1][:N_NODES]], axis=1)[:, :30]
        return o * recip

    def data_agg(p, xx):
        tr = jnp.concatenate([xx, mask], axis=1)
        tr = _linear(p['init_trns'], tr, p['a_init'])
        o1 = prop(p, ev_edges, ev_recip, locs_pad, tr, p['a11'])
        o2 = prop(p, cl_edges, cl_recip, srcs_pad, tr, p['a12'])
        tr1 = _linear(p['l1_t1_2'], jnp.concatenate([tr, o1, mask], axis=1))
        tr2 = _linear(p['l1_t2_2'], jnp.concatenate([tr, o2, mask], axis=1))
        tr = _prelu(jnp.concatenate([tr1, tr2], axis=1), p['a1'])
        h1 = _linear(p['l2_t1_1'], tr, p['a21'])
        h2 = _linear(p['l2_t2_1'], tr, p['a22'])
        o1 = prop(p, ev_edges, ev_recip, locs_pad, h1, jnp.float32(1.0))
        o2 = prop(p, cl_edges, cl_recip, srcs_pad, h2, jnp.float32(1.0))
        tr1 = _linear(p['l2_t1_2'], jnp.concatenate([tr, o1, mask], axis=1))
        tr2 = _linear(p['l2_t2_2'], jnp.concatenate([tr, o2, mask], axis=1))
        return _prelu(jnp.concatenate([tr1, tr2], axis=1), p['a2'])

    for p in params['aggs']:
        x = data_agg(p, x)

    b = params['bip']
    xcat = jnp.concatenate([x, mask], axis=1)
    pos_j_all = locs_cart[A_src_in_sta[0]]
    ej = A_src_in_product[0]
    ei = A_src_in_product[1]
    h = jnp.concatenate([xcat[ej], srcs_cart[ei] - pos_j_all[ej]], axis=1)
    h = _linear(b['fc1b'], _linear(b['fc1a'], h, b['a_fc1']))
    h = _prelu(h, b['a1'])
    agg = jax.ops.segment_sum(h, ei, num_segments=srcs_cart.shape[0])
    out = _linear(b['fc2'], agg, b['a2'])
    pr = params['proj']
    return _linear(pr['l2'], _linear(pr['l1'], out, pr['a']))


# speedup vs baseline: 2.3218x; 1.3009x over previous
import functools

import jax
import jax.numpy as jnp
from jax import lax
from jax.experimental import pallas as pl
from jax.experimental.pallas import tpu as pltpu
from jax.experimental.pallas import tpu_sc as plsc

N_NODES = 100000
ACC_ROWS = 100352
JUNK_ROW = 100000
EB_BLK = 128
NSUB = 16


def _cdiv(a, b):
    return (a + b - 1) // b


@functools.partial(jax.jit, static_argnames=("pre_act", "post_act", "split16"))
def _pl_linear(x, W, b, a_pre, a_post, pre_act=False, post_act=False, split16=False):
    N, K = x.shape
    M = W.shape[1]
    R = 2048
    grid = _cdiv(N, R)

    def body(x_ref, w_ref, b_ref, ap_ref, aq_ref, *out_refs):
        xv = x_ref[...]
        if pre_act:
            ap = ap_ref[0, 0]
            xv = jnp.where(xv >= 0, xv, ap * xv)
        y = jnp.dot(xv, w_ref[...], preferred_element_type=jnp.float32)
        y = y + b_ref[...]
        if post_act:
            aq = aq_ref[0, 0]
            y = jnp.where(y >= 0, y, aq * y)
        if split16:
            out_refs[0][...] = y[:, :16]
            out_refs[1][...] = y[:, 16:32]
        else:
            out_refs[0][...] = y

    if split16:
        out_specs = [pl.BlockSpec((R, 16), lambda i: (i, 0)),
                     pl.BlockSpec((R, 16), lambda i: (i, 0))]
        out_shape = [jax.ShapeDtypeStruct((N, 16), jnp.float32),
                     jax.ShapeDtypeStruct((N, 16), jnp.float32)]
    else:
        out_specs = pl.BlockSpec((R, M), lambda i: (i, 0))
        out_shape = jax.ShapeDtypeStruct((N, M), jnp.float32)

    return pl.pallas_call(
        body,
        grid=(grid,),
        in_specs=[
            pl.BlockSpec((R, K), lambda i: (i, 0)),
            pl.BlockSpec((K, M), lambda i: (0, 0)),
            pl.BlockSpec((1, M), lambda i: (0, 0)),
            pl.BlockSpec((1, 1), lambda i: (0, 0)),
            pl.BlockSpec((1, 1), lambda i: (0, 0)),
        ],
        out_specs=out_specs,
        out_shape=out_shape,
    )(x, W, b.reshape(1, M),
      jnp.asarray(a_pre, jnp.float32).reshape(1, 1),
      jnp.asarray(a_post, jnp.float32).reshape(1, 1))


def _linear(p, x, a=None):
    if a is None:
        return _pl_linear(x, p['W'], p['b'], 0.0, 0.0)
    return _pl_linear(x, p['W'], p['b'], 0.0, a, post_act=True)


def _prelu(x, a):
    return jnp.where(x >= 0, x, a * x)


def _make_sc_prop(E_pad, T, CB):
    CHUNK = CB * 128
    EB = E_pad // NSUB
    NCH = EB // CHUNK
    RPS = EB // 128
    Z = ACC_ROWS // NSUB

    mesh = plsc.VectorSubcoreMesh(core_axis_name="c", subcore_axis_name="s")

    @functools.partial(
        pl.kernel,
        out_type=jax.ShapeDtypeStruct((2, ACC_ROWS, 16), jnp.float32),
        mesh=mesh,
        compiler_params=pltpu.CompilerParams(use_tc_tiling_on_sc=False),
        scratch_types=[
            pltpu.VMEM((T, 16), jnp.float32),
            pltpu.VMEM((2, CB, 128, 16), jnp.float32),
            pltpu.VMEM((2, CB, 128), jnp.int32),
            pltpu.VMEM((2, CB, 128), jnp.int32),
            pltpu.VMEM((2, CB, 128), jnp.int32),
            pltpu.VMEM((2, CB, 128), jnp.int32),
            pltpu.VMEM((16,), jnp.float32),
            pltpu.VMEM_SHARED((ACC_ROWS, 16), jnp.float32),
            pltpu.SemaphoreType.DMA,
            pltpu.SemaphoreType.DMA,
            pltpu.SemaphoreType.DMA,
            pltpu.SemaphoreType.DMA,
        ],
    )
    def sc_prop(u0, u1, g0, g1, src_h, dst_h, ts_h, td_h, a_h, zeros_h, out,
                gtab_v, urows_v, si_v, di_v, ts_v, td_v, a_v, acc,
                gs0, gs1, ss0, ss1):
        c = lax.axis_index("c")
        s = lax.axis_index("s")
        gsem = (gs0, gs1)
        ssem = (ss0, ss1)

        pltpu.sync_copy(zeros_h.at[pl.ds(s * Z, Z)], acc.at[pl.ds(s * Z, Z)])
        pltpu.sync_copy(a_h, a_v)

        @pl.when(c == 0)
        def _():
            pltpu.sync_copy(g0, gtab_v)

        @pl.when(c == 1)
        def _():
            pltpu.sync_copy(g1, gtab_v)

        plsc.subcore_barrier()

        def fetch_idx(buf, row):
            pltpu.sync_copy(src_h.at[pl.ds(row, CB)], si_v.at[buf])
            pltpu.sync_copy(dst_h.at[pl.ds(row, CB)], di_v.at[buf])
            pltpu.sync_copy(ts_h.at[pl.ds(row, CB)], ts_v.at[buf])
            pltpu.sync_copy(td_h.at[pl.ds(row, CB)], td_v.at[buf])

        def fire_gathers(buf):
            @pl.when(c == 0)
            def _():
                for b in range(CB):
                    pltpu.async_copy(u0.at[si_v.at[buf, b]],
                                     urows_v.at[buf, b], gsem[buf])

            @pl.when(c == 1)
            def _():
                for b in range(CB):
                    pltpu.async_copy(u1.at[si_v.at[buf, b]],
                                     urows_v.at[buf, b], gsem[buf])

        def drain_gathers(buf):
            for b in range(CB):
                pltpu.make_async_copy(u0.at[si_v.at[buf, b]],
                                      urows_v.at[buf, b], gsem[buf]).wait()

        def fire_scatters(buf):
            for b in range(CB):
                pltpu.async_copy(urows_v.at[buf, b], acc.at[di_v.at[buf, b]],
                                 ssem[buf], add=True)

        def drain_scatters(buf):
            for b in range(CB):
                pltpu.make_async_copy(urows_v.at[buf, b],
                                      acc.at[di_v.at[buf, b]],
                                      ssem[buf]).wait()

        def compute(buf):
            av = a_v[...]
            for b in range(CB):

                def grp(g, carry, _b=b):
                    ts16 = ts_v[buf, _b, pl.ds(g * 16, 16)]
                    td16 = td_v[buf, _b, pl.ds(g * 16, 16)]
                    for i in range(16):
                        tsi = ts16[i]
                        tdi = td16[i]
                        z = (urows_v[buf, _b, g * 16 + i]
                             + gtab_v[tsi] - gtab_v[tdi])
                        urows_v[buf, _b, g * 16 + i] = (
                            jnp.maximum(z, 0.0) + av * jnp.minimum(z, 0.0))
                    return carry

                lax.fori_loop(0, 8, grp, 0)

        row0 = s * RPS
        fetch_idx(0, row0)
        fire_gathers(0)

        def step(ch2, carry):
            rowA = row0 + ch2 * 2 * CB
            rowB = rowA + CB
            @pl.when(ch2 >= 1)
            def _():
                drain_scatters(1)

            fetch_idx(1, rowB)
            fire_gathers(1)
            drain_gathers(0)
            compute(0)
            fire_scatters(0)
            drain_scatters(0)

            @pl.when(ch2 <= NCH // 2 - 2)
            def _():
                fetch_idx(0, rowB + CB)
                fire_gathers(0)

            drain_gathers(1)
            compute(1)
            fire_scatters(1)
            return carry

        lax.fori_loop(0, NCH // 2, step, 0)
        drain_scatters(1)
        plsc.subcore_barrier()

        @pl.when(c == 0)
        def _():
            pltpu.sync_copy(acc.at[pl.ds(s * Z, Z)], out.at[0, pl.ds(s * Z, Z)])

        @pl.when(c == 1)
        def _():
            pltpu.sync_copy(acc.at[pl.ds(s * Z, Z)], out.at[1, pl.ds(s * Z, Z)])

    return sc_prop


def _make_sc_count(E_pad):
    RPS = E_pad // NSUB // 128
    Z = ACC_ROWS // NSUB
    mesh = plsc.VectorSubcoreMesh(core_axis_name="c", subcore_axis_name="s")

    @functools.partial(
        pl.kernel,
        out_type=jax.ShapeDtypeStruct((2, ACC_ROWS, 16), jnp.float32),
        mesh=mesh,
        compiler_params=pltpu.CompilerParams(use_tc_tiling_on_sc=False),
        scratch_types=[
            pltpu.VMEM((EB_BLK, 16), jnp.float32),
            pltpu.VMEM((2, 1, EB_BLK), jnp.int32),
            pltpu.VMEM_SHARED((ACC_ROWS, 16), jnp.float32),
            pltpu.SemaphoreType.DMA,
            pltpu.SemaphoreType.DMA,
        ],
    )
    def sc_count(dst_h, ones_h, zeros_h, out, ones_v, di_v, acc, ss0, ss1):
        c = lax.axis_index("c")
        s = lax.axis_index("s")
        ssem = (ss0, ss1)
        pltpu.sync_copy(zeros_h.at[pl.ds(s * Z, Z)], acc.at[pl.ds(s * Z, Z)])
        pltpu.sync_copy(ones_h, ones_v)
        plsc.subcore_barrier()
        row0 = s * RPS

        def blk(j2, carry):
            for buf in range(2):
                @pl.when(j2 >= 1)
                def _(_buf=buf):
                    pltpu.make_async_copy(
                        ones_v, acc.at[di_v.at[_buf, 0]],
                        ssem[_buf]).wait()

                pltpu.sync_copy(dst_h.at[pl.ds(row0 + j2 * 2 + buf, 1)],
                                di_v.at[buf])
                pltpu.async_copy(ones_v, acc.at[di_v.at[buf, 0]],
                                 ssem[buf], add=True)
            return carry

        lax.fori_loop(0, RPS // 2, blk, 0)
        for buf in range(2):
            pltpu.make_async_copy(ones_v, acc.at[di_v.at[buf, 0]],
                                  ssem[buf]).wait()
        plsc.subcore_barrier()

        @pl.when(c == 0)
        def _():
            pltpu.sync_copy(acc.at[pl.ds(s * Z, Z)], out.at[0, pl.ds(s * Z, Z)])

        @pl.when(c == 1)
        def _():
            pltpu.sync_copy(acc.at[pl.ds(s * Z, Z)], out.at[1, pl.ds(s * Z, Z)])

    return sc_count


def _pad_edges(v, E_pad, fill):
    E = v.shape[0]
    return jnp.pad(v, (0, E_pad - E), constant_values=fill)


def kernel(x, mask, A_in_events, A_in_clusters, A_src_in_product, A_src_in_sta,
           locs_cart, srcs_cart, params):
    n = x.shape[0]
    rel = locs_cart[A_src_in_sta[0]] - srcs_cart[A_src_in_sta[1]]
    x = jnp.concatenate([x, rel], axis=1)
    mask = jnp.concatenate([mask, rel], axis=1)
    e = params['embed']
    mask = _linear(e['l2'], _linear(e['l1'], mask, e['a']))
    sta_of_node = A_src_in_sta[0].astype(jnp.int32)
    src_of_node = A_src_in_sta[1].astype(jnp.int32)

    E = A_in_events.shape[1]
    E_pad = _cdiv(E, NSUB * 1024) * NSUB * 1024
    sc_prop_ev = _make_sc_prop(E_pad, 512, 4)
    sc_prop_cl = _make_sc_prop(E_pad, 1024, 2)
    sc_count = _make_sc_count(E_pad)

    zeros_h = jnp.zeros((ACC_ROWS, 16), jnp.float32)
    ones_h = jnp.ones((EB_BLK, 16), jnp.float32)

    def prep_edges(A, tbl_of_node):
        src = _pad_edges(A[0].astype(jnp.int32), E_pad, 0).reshape(-1, 128)
        dst = _pad_edges(A[1].astype(jnp.int32), E_pad, JUNK_ROW).reshape(-1, 128)
        ts = _pad_edges(tbl_of_node[A[0]], E_pad, 0).reshape(-1, 128)
        td = _pad_edges(tbl_of_node[A[1]], E_pad, 0).reshape(-1, 128)
        cnt = sc_count(dst, ones_h, zeros_h)[0][:N_NODES, :1]
        recip = 1.0 / jnp.maximum(cnt, 1.0)
        return (src, dst, ts, td), recip

    ev_edges, ev_recip = prep_edges(A_in_events, sta_of_node)
    cl_edges, cl_recip = prep_edges(A_in_clusters, src_of_node)

    locs_pad = jnp.pad(locs_cart, ((0, 512 - locs_cart.shape[0]), (0, 0)))
    srcs_pad = jnp.pad(srcs_cart, ((0, 1024 - srcs_cart.shape[0]), (0, 0)))

    def prop(sc_prop, p, edges, recip, base_tab, xin_pre, a_in):
        Wm, bm = p['merge']['W'], p['merge']['b']
        W_node = jnp.pad(Wm[:30], ((0, 0), (0, 2)))
        b_node = jnp.pad(bm, (0, 2))
        u0, u1 = _pl_linear(xin_pre, W_node, b_node, a_in, 0.0,
                            pre_act=True, split16=True)
        gt = base_tab @ jnp.pad(Wm[30:34], ((0, 0), (0, 2)))
        gt0, gt1 = gt[:, :16], gt[:, 16:32]
        src, dst, ts, td = edges
        a_vec = jnp.full((16,), p['a_merge'], jnp.float32)
        out = sc_prop(u0, u1, gt0, gt1, src, dst, ts, td, a_vec, zeros_h)
        o = jnp.concatenate([out[0][:N_NODES], out[1][:N_NODES]], axis=1)[:, :30]
        return o * recip

    def data_agg(p, xx):
        tr = jnp.concatenate([xx, mask], axis=1)
        tr = _linear(p['init_trns'], tr, p['a_init'])
        o1 = prop(sc_prop_ev, p, ev_edges, ev_recip, locs_pad, tr, p['a11'])
        o2 = prop(sc_prop_cl, p, cl_edges, cl_recip, srcs_pad, tr, p['a12'])
        tr1 = _linear(p['l1_t1_2'], jnp.concatenate([tr, o1, mask], axis=1))
        tr2 = _linear(p['l1_t2_2'], jnp.concatenate([tr, o2, mask], axis=1))
        tr = _prelu(jnp.concatenate([tr1, tr2], axis=1), p['a1'])
        h1 = _linear(p['l2_t1_1'], tr, p['a21'])
        h2 = _linear(p['l2_t2_1'], tr, p['a22'])
        o1 = prop(sc_prop_ev, p, ev_edges, ev_recip, locs_pad, h1, jnp.float32(1.0))
        o2 = prop(sc_prop_cl, p, cl_edges, cl_recip, srcs_pad, h2, jnp.float32(1.0))
        tr1 = _linear(p['l2_t1_2'], jnp.concatenate([tr, o1, mask], axis=1))
        tr2 = _linear(p['l2_t2_2'], jnp.concatenate([tr, o2, mask], axis=1))
        return _prelu(jnp.concatenate([tr1, tr2], axis=1), p['a2'])

    for p in params['aggs']:
        x = data_agg(p, x)

    b = params['bip']
    xcat = jnp.concatenate([x, mask], axis=1)
    pos_j_all = locs_cart[A_src_in_sta[0]]
    ej = A_src_in_product[0]
    ei = A_src_in_product[1]
    h = jnp.concatenate([xcat[ej], srcs_cart[ei] - pos_j_all[ej]], axis=1)
    h = _linear(b['fc1b'], _linear(b['fc1a'], h, b['a_fc1']))
    h = _prelu(h, b['a1'])
    agg = jax.ops.segment_sum(h, ei, num_segments=srcs_cart.shape[0])
    out = _linear(b['fc2'], agg, b['a2'])
    pr = params['proj']
    return _linear(pr['l2'], _linear(pr['l1'], out, pr['a']))


# speedup vs baseline: 2.3359x; 1.0061x over previous
import functools

import jax
import jax.numpy as jnp
from jax import lax
from jax.experimental import pallas as pl
from jax.experimental.pallas import tpu as pltpu
from jax.experimental.pallas import tpu_sc as plsc

N_NODES = 100000
ACC_ROWS = 100352
JUNK_ROW = 100000
EB_BLK = 128
NSUB = 16


def _cdiv(a, b):
    return (a + b - 1) // b


@functools.partial(jax.jit, static_argnames=("pre_act", "post_act", "split16"))
def _pl_linear(x, W, b, a_pre, a_post, pre_act=False, post_act=False, split16=False):
    N, K = x.shape
    M = W.shape[1]
    R = 2048
    grid = _cdiv(N, R)

    def body(x_ref, w_ref, b_ref, ap_ref, aq_ref, *out_refs):
        xv = x_ref[...]
        if pre_act:
            ap = ap_ref[0, 0]
            xv = jnp.where(xv >= 0, xv, ap * xv)
        y = jnp.dot(xv, w_ref[...], preferred_element_type=jnp.float32)
        y = y + b_ref[...]
        if post_act:
            aq = aq_ref[0, 0]
            y = jnp.where(y >= 0, y, aq * y)
        if split16:
            out_refs[0][...] = y[:, :16]
            out_refs[1][...] = y[:, 16:32]
        else:
            out_refs[0][...] = y

    if split16:
        out_specs = [pl.BlockSpec((R, 16), lambda i: (i, 0)),
                     pl.BlockSpec((R, 16), lambda i: (i, 0))]
        out_shape = [jax.ShapeDtypeStruct((N, 16), jnp.float32),
                     jax.ShapeDtypeStruct((N, 16), jnp.float32)]
    else:
        out_specs = pl.BlockSpec((R, M), lambda i: (i, 0))
        out_shape = jax.ShapeDtypeStruct((N, M), jnp.float32)

    return pl.pallas_call(
        body,
        grid=(grid,),
        in_specs=[
            pl.BlockSpec((R, K), lambda i: (i, 0)),
            pl.BlockSpec((K, M), lambda i: (0, 0)),
            pl.BlockSpec((1, M), lambda i: (0, 0)),
            pl.BlockSpec((1, 1), lambda i: (0, 0)),
            pl.BlockSpec((1, 1), lambda i: (0, 0)),
        ],
        out_specs=out_specs,
        out_shape=out_shape,
    )(x, W, b.reshape(1, M),
      jnp.asarray(a_pre, jnp.float32).reshape(1, 1),
      jnp.asarray(a_post, jnp.float32).reshape(1, 1))


@functools.partial(jax.jit, static_argnames=("post_act",))
def _pl_linear_pair(x0, x1, W, b, a_post, post_act=True):
    N = x0.shape[0]
    M = W.shape[1]
    R = 2048
    grid = _cdiv(N, R)

    def body(x0_ref, x1_ref, w_ref, b_ref, aq_ref, o0_ref, o1_ref):
        y = jnp.dot(x0_ref[...], w_ref[:16, :],
                    preferred_element_type=jnp.float32)
        y = y + jnp.dot(x1_ref[...], w_ref[16:32, :],
                        preferred_element_type=jnp.float32)
        y = y + b_ref[...]
        if post_act:
            aq = aq_ref[0, 0]
            y = jnp.where(y >= 0, y, aq * y)
        o0_ref[...] = y[:, :16]
        o1_ref[...] = y[:, 16:32]

    return pl.pallas_call(
        body,
        grid=(grid,),
        in_specs=[
            pl.BlockSpec((R, 16), lambda i: (i, 0)),
            pl.BlockSpec((R, 16), lambda i: (i, 0)),
            pl.BlockSpec((32, M), lambda i: (0, 0)),
            pl.BlockSpec((1, M), lambda i: (0, 0)),
            pl.BlockSpec((1, 1), lambda i: (0, 0)),
        ],
        out_specs=[pl.BlockSpec((R, 16), lambda i: (i, 0)),
                   pl.BlockSpec((R, 16), lambda i: (i, 0))],
        out_shape=[jax.ShapeDtypeStruct((N, 16), jnp.float32),
                   jax.ShapeDtypeStruct((N, 16), jnp.float32)],
    )(x0, x1, W, b.reshape(1, M),
      jnp.asarray(a_post, jnp.float32).reshape(1, 1))


def _linear(p, x, a=None):
    if a is None:
        return _pl_linear(x, p['W'], p['b'], 0.0, 0.0)
    return _pl_linear(x, p['W'], p['b'], 0.0, a, post_act=True)


def _prelu(x, a):
    return jnp.where(x >= 0, x, a * x)


def _make_sc_prop(E_pad, T, CB):
    CHUNK = CB * 128
    EB = E_pad // NSUB
    NCH = EB // CHUNK
    RPS = EB // 128
    Z = ACC_ROWS // NSUB

    mesh = plsc.VectorSubcoreMesh(core_axis_name="c", subcore_axis_name="s")

    @functools.partial(
        pl.kernel,
        out_type=jax.ShapeDtypeStruct((2, ACC_ROWS, 16), jnp.float32),
        mesh=mesh,
        compiler_params=pltpu.CompilerParams(use_tc_tiling_on_sc=False),
        scratch_types=[
            pltpu.VMEM((T, 16), jnp.float32),
            pltpu.VMEM((2, CB, 128, 16), jnp.float32),
            pltpu.VMEM((2, CB, 128), jnp.int32),
            pltpu.VMEM((2, CB, 128), jnp.int32),
            pltpu.VMEM((2, CB, 128), jnp.int32),
            pltpu.VMEM((2, CB, 128), jnp.int32),
            pltpu.VMEM((16,), jnp.float32),
            pltpu.VMEM_SHARED((ACC_ROWS, 16), jnp.float32),
            pltpu.SemaphoreType.DMA,
            pltpu.SemaphoreType.DMA,
            pltpu.SemaphoreType.DMA,
            pltpu.SemaphoreType.DMA,
        ],
    )
    def sc_prop(u0, u1, g0, g1, src_h, dst_h, ts_h, td_h, a_h, zeros_h, out,
                gtab_v, urows_v, si_v, di_v, ts_v, td_v, a_v, acc,
                gs0, gs1, ss0, ss1):
        c = lax.axis_index("c")
        s = lax.axis_index("s")
        gsem = (gs0, gs1)
        ssem = (ss0, ss1)

        pltpu.sync_copy(zeros_h.at[pl.ds(s * Z, Z)], acc.at[pl.ds(s * Z, Z)])
        pltpu.sync_copy(a_h, a_v)

        @pl.when(c == 0)
        def _():
            pltpu.sync_copy(g0, gtab_v)

        @pl.when(c == 1)
        def _():
            pltpu.sync_copy(g1, gtab_v)

        plsc.subcore_barrier()

        def fetch_idx(buf, row):
            pltpu.sync_copy(src_h.at[pl.ds(row, CB)], si_v.at[buf])
            pltpu.sync_copy(dst_h.at[pl.ds(row, CB)], di_v.at[buf])
            pltpu.sync_copy(ts_h.at[pl.ds(row, CB)], ts_v.at[buf])
            pltpu.sync_copy(td_h.at[pl.ds(row, CB)], td_v.at[buf])

        def fire_gathers(buf):
            @pl.when(c == 0)
            def _():
                for b in range(CB):
                    pltpu.async_copy(u0.at[si_v.at[buf, b]],
                                     urows_v.at[buf, b], gsem[buf])

            @pl.when(c == 1)
            def _():
                for b in range(CB):
                    pltpu.async_copy(u1.at[si_v.at[buf, b]],
                                     urows_v.at[buf, b], gsem[buf])

        def drain_gathers(buf):
            for b in range(CB):
                pltpu.make_async_copy(u0.at[si_v.at[buf, b]],
                                      urows_v.at[buf, b], gsem[buf]).wait()

        def fire_scatters(buf):
            for b in range(CB):
                pltpu.async_copy(urows_v.at[buf, b], acc.at[di_v.at[buf, b]],
                                 ssem[buf], add=True)

        def drain_scatters(buf):
            for b in range(CB):
                pltpu.make_async_copy(urows_v.at[buf, b],
                                      acc.at[di_v.at[buf, b]],
                                      ssem[buf]).wait()

        def compute(buf):
            av = a_v[...]
            for b in range(CB):

                def grp(g, carry, _b=b):
                    ts16 = ts_v[buf, _b, pl.ds(g * 16, 16)]
                    td16 = td_v[buf, _b, pl.ds(g * 16, 16)]
                    for i in range(16):
                        tsi = ts16[i]
                        tdi = td16[i]
                        z = (urows_v[buf, _b, g * 16 + i]
                             + gtab_v[tsi] - gtab_v[tdi])
                        urows_v[buf, _b, g * 16 + i] = (
                            jnp.maximum(z, 0.0) + av * jnp.minimum(z, 0.0))
                    return carry

                lax.fori_loop(0, 8, grp, 0)

        row0 = s * RPS
        fetch_idx(0, row0)
        fire_gathers(0)

        def step(ch2, carry):
            rowA = row0 + ch2 * 2 * CB
            rowB = rowA + CB
            @pl.when(ch2 >= 1)
            def _():
                drain_scatters(1)

            fetch_idx(1, rowB)
            fire_gathers(1)
            drain_gathers(0)
            compute(0)
            fire_scatters(0)
            drain_scatters(0)

            @pl.when(ch2 <= NCH // 2 - 2)
            def _():
                fetch_idx(0, rowB + CB)
                fire_gathers(0)

            drain_gathers(1)
            compute(1)
            fire_scatters(1)
            return carry

        lax.fori_loop(0, NCH // 2, step, 0)
        drain_scatters(1)
        plsc.subcore_barrier()

        @pl.when(c == 0)
        def _():
            pltpu.sync_copy(acc.at[pl.ds(s * Z, Z)], out.at[0, pl.ds(s * Z, Z)])

        @pl.when(c == 1)
        def _():
            pltpu.sync_copy(acc.at[pl.ds(s * Z, Z)], out.at[1, pl.ds(s * Z, Z)])

    return sc_prop


def _make_sc_count(E_pad):
    RPS = E_pad // NSUB // 128
    Z = ACC_ROWS // NSUB
    mesh = plsc.VectorSubcoreMesh(core_axis_name="c", subcore_axis_name="s")

    @functools.partial(
        pl.kernel,
        out_type=jax.ShapeDtypeStruct((2, ACC_ROWS, 16), jnp.float32),
        mesh=mesh,
        compiler_params=pltpu.CompilerParams(use_tc_tiling_on_sc=False),
        scratch_types=[
            pltpu.VMEM((EB_BLK, 16), jnp.float32),
            pltpu.VMEM((2, 1, EB_BLK), jnp.int32),
            pltpu.VMEM_SHARED((ACC_ROWS, 16), jnp.float32),
            pltpu.SemaphoreType.DMA,
            pltpu.SemaphoreType.DMA,
        ],
    )
    def sc_count(dst_h, ones_h, zeros_h, out, ones_v, di_v, acc, ss0, ss1):
        c = lax.axis_index("c")
        s = lax.axis_index("s")
        ssem = (ss0, ss1)
        pltpu.sync_copy(zeros_h.at[pl.ds(s * Z, Z)], acc.at[pl.ds(s * Z, Z)])
        pltpu.sync_copy(ones_h, ones_v)
        plsc.subcore_barrier()
        row0 = s * RPS

        def blk(j2, carry):
            for buf in range(2):
                @pl.when(j2 >= 1)
                def _(_buf=buf):
                    pltpu.make_async_copy(
                        ones_v, acc.at[di_v.at[_buf, 0]],
                        ssem[_buf]).wait()

                pltpu.sync_copy(dst_h.at[pl.ds(row0 + j2 * 2 + buf, 1)],
                                di_v.at[buf])
                pltpu.async_copy(ones_v, acc.at[di_v.at[buf, 0]],
                                 ssem[buf], add=True)
            return carry

        lax.fori_loop(0, RPS // 2, blk, 0)
        for buf in range(2):
            pltpu.make_async_copy(ones_v, acc.at[di_v.at[buf, 0]],
                                  ssem[buf]).wait()
        plsc.subcore_barrier()

        @pl.when(c == 0)
        def _():
            pltpu.sync_copy(acc.at[pl.ds(s * Z, Z)], out.at[0, pl.ds(s * Z, Z)])

        @pl.when(c == 1)
        def _():
            pltpu.sync_copy(acc.at[pl.ds(s * Z, Z)], out.at[1, pl.ds(s * Z, Z)])

    return sc_count


def _make_sc_edge_mlp(E_pad, T, CB):
    CHUNK = CB * 128
    EB = E_pad // NSUB
    NCH = EB // CHUNK
    RPS = EB // 128
    mesh = plsc.VectorSubcoreMesh(core_axis_name="c", subcore_axis_name="s")

    @functools.partial(
        pl.kernel,
        out_type=jax.ShapeDtypeStruct((2, E_pad // 128, 128, 16), jnp.float32),
        mesh=mesh,
        compiler_params=pltpu.CompilerParams(use_tc_tiling_on_sc=False),
        scratch_types=[
            pltpu.VMEM((T, 16), jnp.float32),
            pltpu.VMEM((2, CB, 128, 16), jnp.float32),
            pltpu.VMEM((2, CB, 128), jnp.int32),
            pltpu.VMEM((2, CB, 128), jnp.int32),
            pltpu.VMEM((2, CB, 128), jnp.int32),
            pltpu.VMEM((16,), jnp.float32),
            pltpu.SemaphoreType.DMA,
            pltpu.SemaphoreType.DMA,
            pltpu.SemaphoreType.DMA,
            pltpu.SemaphoreType.DMA,
        ],
    )
    def sc_emlp(p0, p1, g0, g1, ej_h, ts_h, td_h, a_h, out,
                gtab_v, urows_v, si_v, ts_v, td_v, a_v, gs0, gs1, ws0, ws1):
        c = lax.axis_index("c")
        s = lax.axis_index("s")
        gsem = (gs0, gs1)
        wsem = (ws0, ws1)
        pltpu.sync_copy(a_h, a_v)

        @pl.when(c == 0)
        def _():
            pltpu.sync_copy(g0, gtab_v)

        @pl.when(c == 1)
        def _():
            pltpu.sync_copy(g1, gtab_v)

        def fetch_idx(buf, row):
            pltpu.sync_copy(ej_h.at[pl.ds(row, CB)], si_v.at[buf])
            pltpu.sync_copy(ts_h.at[pl.ds(row, CB)], ts_v.at[buf])
            pltpu.sync_copy(td_h.at[pl.ds(row, CB)], td_v.at[buf])

        def fire_gathers(buf):
            @pl.when(c == 0)
            def _():
                for b in range(CB):
                    pltpu.async_copy(p0.at[si_v.at[buf, b]],
                                     urows_v.at[buf, b], gsem[buf])

            @pl.when(c == 1)
            def _():
                for b in range(CB):
                    pltpu.async_copy(p1.at[si_v.at[buf, b]],
                                     urows_v.at[buf, b], gsem[buf])

        def drain_gathers(buf):
            for b in range(CB):
                pltpu.make_async_copy(p0.at[si_v.at[buf, b]],
                                      urows_v.at[buf, b], gsem[buf]).wait()

        def fire_writes(buf, row):
            @pl.when(c == 0)
            def _():
                pltpu.async_copy(urows_v.at[buf], out.at[0, pl.ds(row, CB)],
                                 wsem[buf])

            @pl.when(c == 1)
            def _():
                pltpu.async_copy(urows_v.at[buf], out.at[1, pl.ds(row, CB)],
                                 wsem[buf])

        def drain_writes(buf, row):
            pltpu.make_async_copy(urows_v.at[buf], out.at[0, pl.ds(row, CB)],
                                  wsem[buf]).wait()

        def compute(buf):
            av = a_v[...]
            for b in range(CB):

                def grp(g, carry, _b=b):
                    ts16 = ts_v[buf, _b, pl.ds(g * 16, 16)]
                    td16 = td_v[buf, _b, pl.ds(g * 16, 16)]
                    for i in range(16):
                        tsi = ts16[i]
                        tdi = td16[i]
                        z = (urows_v[buf, _b, g * 16 + i]
                             + gtab_v[tsi] - gtab_v[tdi])
                        urows_v[buf, _b, g * 16 + i] = (
                            jnp.maximum(z, 0.0) + av * jnp.minimum(z, 0.0))
                    return carry

                lax.fori_loop(0, 8, grp, 0)

        row0 = s * RPS
        fetch_idx(0, row0)
        fire_gathers(0)

        def step(ch2, carry):
            rowA = row0 + ch2 * 2 * CB
            rowB = rowA + CB

            @pl.when(ch2 >= 1)
            def _():
                drain_writes(1, rowB - 2 * CB)

            fetch_idx(1, rowB)
            fire_gathers(1)
            drain_gathers(0)
            compute(0)
            fire_writes(0, rowA)
            drain_writes(0, rowA)

            @pl.when(ch2 <= NCH // 2 - 2)
            def _():
                fetch_idx(0, rowB + CB)
                fire_gathers(0)

            drain_gathers(1)
            compute(1)
            fire_writes(1, rowB)
            return carry

        lax.fori_loop(0, NCH // 2, step, 0)
        drain_writes(1, row0 + (NCH - 1) * CB)

    return sc_emlp


SEG_ROWS = 1024
SEG_JUNK = 1000


def _make_sc_seg_sum(E_pad):
    RPS = E_pad // NSUB // 128
    Z = SEG_ROWS // NSUB
    mesh = plsc.VectorSubcoreMesh(core_axis_name="c", subcore_axis_name="s")

    @functools.partial(
        pl.kernel,
        out_type=jax.ShapeDtypeStruct((2, SEG_ROWS, 16), jnp.float32),
        mesh=mesh,
        compiler_params=pltpu.CompilerParams(use_tc_tiling_on_sc=False),
        scratch_types=[
            pltpu.VMEM((2, 1, 128, 16), jnp.float32),
            pltpu.VMEM((2, 1, 128), jnp.int32),
            pltpu.VMEM_SHARED((SEG_ROWS, 16), jnp.float32),
            pltpu.SemaphoreType.DMA,
            pltpu.SemaphoreType.DMA,
        ],
    )
    def sc_seg(h0, h1, ei_h, zeros_h, out, pay_v, di_v, acc, ss0, ss1):
        c = lax.axis_index("c")
        s = lax.axis_index("s")
        ssem = (ss0, ss1)
        pltpu.sync_copy(zeros_h.at[pl.ds(s * Z, Z)], acc.at[pl.ds(s * Z, Z)])
        plsc.subcore_barrier()
        row0 = s * RPS

        def blk(j2, carry):
            for buf in range(2):
                @pl.when(j2 >= 1)
                def _(_buf=buf):
                    pltpu.make_async_copy(pay_v.at[_buf, 0],
                                          acc.at[di_v.at[_buf, 0]],
                                          ssem[_buf]).wait()

                row = row0 + j2 * 2 + buf
                pltpu.sync_copy(ei_h.at[pl.ds(row, 1)], di_v.at[buf])

                @pl.when(c == 0)
                def _(_buf=buf, _row=row):
                    pltpu.sync_copy(h0.at[pl.ds(_row, 1)], pay_v.at[_buf])

                @pl.when(c == 1)
                def _(_buf=buf, _row=row):
                    pltpu.sync_copy(h1.at[pl.ds(_row, 1)], pay_v.at[_buf])

                pltpu.async_copy(pay_v.at[buf, 0], acc.at[di_v.at[buf, 0]],
                                 ssem[buf], add=True)
            return carry

        lax.fori_loop(0, RPS // 2, blk, 0)
        for buf in range(2):
            pltpu.make_async_copy(pay_v.at[buf, 0], acc.at[di_v.at[buf, 0]],
                                  ssem[buf]).wait()
        plsc.subcore_barrier()

        @pl.when(c == 0)
        def _():
            pltpu.sync_copy(acc.at[pl.ds(s * Z, Z)], out.at[0, pl.ds(s * Z, Z)])

        @pl.when(c == 1)
        def _():
            pltpu.sync_copy(acc.at[pl.ds(s * Z, Z)], out.at[1, pl.ds(s * Z, Z)])

    return sc_seg


def _pad_edges(v, E_pad, fill):
    E = v.shape[0]
    return jnp.pad(v, (0, E_pad - E), constant_values=fill)


def kernel(x, mask, A_in_events, A_in_clusters, A_src_in_product, A_src_in_sta,
           locs_cart, srcs_cart, params):
    n = x.shape[0]
    rel = locs_cart[A_src_in_sta[0]] - srcs_cart[A_src_in_sta[1]]
    x = jnp.concatenate([x, rel], axis=1)
    mask = jnp.concatenate([mask, rel], axis=1)
    e = params['embed']
    mask = _linear(e['l2'], _linear(e['l1'], mask, e['a']))
    sta_of_node = A_src_in_sta[0].astype(jnp.int32)
    src_of_node = A_src_in_sta[1].astype(jnp.int32)

    E = A_in_events.shape[1]
    E_pad = _cdiv(E, NSUB * 1024) * NSUB * 1024
    sc_prop_ev = _make_sc_prop(E_pad, 512, 4)
    sc_prop_cl = _make_sc_prop(E_pad, 1024, 2)
    sc_count = _make_sc_count(E_pad)

    zeros_h = jnp.zeros((ACC_ROWS, 16), jnp.float32)
    ones_h = jnp.ones((EB_BLK, 16), jnp.float32)

    def prep_edges(A, tbl_of_node):
        src = _pad_edges(A[0].astype(jnp.int32), E_pad, 0).reshape(-1, 128)
        dst = _pad_edges(A[1].astype(jnp.int32), E_pad, JUNK_ROW).reshape(-1, 128)
        ts = _pad_edges(tbl_of_node[A[0]], E_pad, 0).reshape(-1, 128)
        td = _pad_edges(tbl_of_node[A[1]], E_pad, 0).reshape(-1, 128)
        cnt = sc_count(dst, ones_h, zeros_h)[0][:N_NODES, :1]
        recip = 1.0 / jnp.maximum(cnt, 1.0)
        return (src, dst, ts, td), recip

    ev_edges, ev_recip = prep_edges(A_in_events, sta_of_node)
    cl_edges, cl_recip = prep_edges(A_in_clusters, src_of_node)

    locs_pad = jnp.pad(locs_cart, ((0, 512 - locs_cart.shape[0]), (0, 0)))
    srcs_pad = jnp.pad(srcs_cart, ((0, 1024 - srcs_cart.shape[0]), (0, 0)))

    def prop(sc_prop, p, edges, recip, base_tab, xin_pre, a_in):
        Wm, bm = p['merge']['W'], p['merge']['b']
        W_node = jnp.pad(Wm[:30], ((0, 0), (0, 2)))
        b_node = jnp.pad(bm, (0, 2))
        u0, u1 = _pl_linear(xin_pre, W_node, b_node, a_in, 0.0,
                            pre_act=True, split16=True)
        gt = base_tab @ jnp.pad(Wm[30:34], ((0, 0), (0, 2)))
        gt0, gt1 = gt[:, :16], gt[:, 16:32]
        src, dst, ts, td = edges
        a_vec = jnp.full((16,), p['a_merge'], jnp.float32)
        out = sc_prop(u0, u1, gt0, gt1, src, dst, ts, td, a_vec, zeros_h)
        o = jnp.concatenate([out[0][:N_NODES], out[1][:N_NODES]], axis=1)[:, :30]
        return o * recip

    def data_agg(p, xx):
        tr = jnp.concatenate([xx, mask], axis=1)
        tr = _linear(p['init_trns'], tr, p['a_init'])
        o1 = prop(sc_prop_ev, p, ev_edges, ev_recip, locs_pad, tr, p['a11'])
        o2 = prop(sc_prop_cl, p, cl_edges, cl_recip, srcs_pad, tr, p['a12'])
        tr1 = _linear(p['l1_t1_2'], jnp.concatenate([tr, o1, mask], axis=1))
        tr2 = _linear(p['l1_t2_2'], jnp.concatenate([tr, o2, mask], axis=1))
        tr = _prelu(jnp.concatenate([tr1, tr2], axis=1), p['a1'])
        h1 = _linear(p['l2_t1_1'], tr, p['a21'])
        h2 = _linear(p['l2_t2_1'], tr, p['a22'])
        o1 = prop(sc_prop_ev, p, ev_edges, ev_recip, locs_pad, h1, jnp.float32(1.0))
        o2 = prop(sc_prop_cl, p, cl_edges, cl_recip, srcs_pad, h2, jnp.float32(1.0))
        tr1 = _linear(p['l2_t1_2'], jnp.concatenate([tr, o1, mask], axis=1))
        tr2 = _linear(p['l2_t2_2'], jnp.concatenate([tr, o2, mask], axis=1))
        return _prelu(jnp.concatenate([tr1, tr2], axis=1), p['a2'])

    for p in params['aggs']:
        x = data_agg(p, x)

    b = params['bip']
    xcat = jnp.concatenate([x, mask], axis=1)
    ej = A_src_in_product[0].astype(jnp.int32)
    ei = A_src_in_product[1].astype(jnp.int32)
    sta_ej = sta_of_node[ej]
    W1 = b['fc1a']['W']
    W1x = jnp.pad(W1[:40], ((0, 0), (0, 2)))
    W1p = jnp.pad(W1[40:44], ((0, 0), (0, 2)))
    b1 = jnp.pad(b['fc1a']['b'], (0, 2))
    P00, P01 = _pl_linear(xcat, W1x, b1, 0.0, 0.0, split16=True)
    lt = jnp.pad(locs_cart @ W1p, ((0, 12), (0, 0)))
    Q = jnp.pad(srcs_cart @ W1p, ((0, 24), (0, 0)))
    bigtab = jnp.concatenate([-lt, -Q], axis=0)

    E2 = ej.shape[0]
    E_pad2 = _cdiv(E2, NSUB * 1024) * NSUB * 1024
    sc_emlp = _make_sc_edge_mlp(E_pad2, 1536, 4)
    sc_seg = _make_sc_seg_sum(E_pad2)

    ej_p = _pad_edges(ej, E_pad2, 0).reshape(-1, 128)
    ts_p = _pad_edges(sta_ej, E_pad2, 0).reshape(-1, 128)
    td_p = _pad_edges(512 + ei, E_pad2, 512 + SEG_JUNK).reshape(-1, 128)
    ei_p = _pad_edges(ei, E_pad2, SEG_JUNK).reshape(-1, 128)
    a_fc1_vec = jnp.full((16,), b['a_fc1'], jnp.float32)
    Y = sc_emlp(P00, P01, bigtab[:, :16], bigtab[:, 16:32],
                ej_p, ts_p, td_p, a_fc1_vec)
    Y0 = Y[0].reshape(E_pad2, 16)
    Y1 = Y[1].reshape(E_pad2, 16)
    W2 = jnp.pad(b['fc1b']['W'], ((0, 2), (0, 2)))
    b2 = jnp.pad(b['fc1b']['b'], (0, 2))
    H0, H1 = _pl_linear_pair(Y0, Y1, W2, b2, b['a1'])
    seg = sc_seg(H0.reshape(-1, 128, 16), H1.reshape(-1, 128, 16),
                 ei_p, zeros_h[:SEG_ROWS])
    nsrc = srcs_cart.shape[0]
    agg = jnp.concatenate([seg[0][:nsrc], seg[1][:nsrc]], axis=1)[:, :30]
    out = _linear(b['fc2'], agg, b['a2'])
    pr = params['proj']
    return _linear(pr['l2'], _linear(pr['l1'], out, pr['a']))


# speedup vs baseline: 2.5240x; 1.0805x over previous
import functools

import jax
import jax.numpy as jnp
from jax import lax
from jax.experimental import pallas as pl
from jax.experimental.pallas import tpu as pltpu
from jax.experimental.pallas import tpu_sc as plsc

N_NODES = 100000
ACC_ROWS = 100352
JUNK_ROW = 100000
EB_BLK = 128
NSUB = 16


def _cdiv(a, b):
    return (a + b - 1) // b


@functools.partial(jax.jit, static_argnames=("pre_act", "post_act", "split16"))
def _pl_linear(x, W, b, a_pre, a_post, pre_act=False, post_act=False, split16=False):
    N, K = x.shape
    M = W.shape[1]
    R = 2048
    grid = _cdiv(N, R)

    def body(x_ref, w_ref, b_ref, ap_ref, aq_ref, *out_refs):
        xv = x_ref[...]
        if pre_act:
            ap = ap_ref[0, 0]
            xv = jnp.where(xv >= 0, xv, ap * xv)
        y = jnp.dot(xv, w_ref[...], preferred_element_type=jnp.float32)
        y = y + b_ref[...]
        if post_act:
            aq = aq_ref[0, 0]
            y = jnp.where(y >= 0, y, aq * y)
        if split16:
            out_refs[0][...] = y[:, :16]
            out_refs[1][...] = y[:, 16:32]
        else:
            out_refs[0][...] = y

    if split16:
        out_specs = [pl.BlockSpec((R, 16), lambda i: (i, 0)),
                     pl.BlockSpec((R, 16), lambda i: (i, 0))]
        out_shape = [jax.ShapeDtypeStruct((N, 16), jnp.float32),
                     jax.ShapeDtypeStruct((N, 16), jnp.float32)]
    else:
        out_specs = pl.BlockSpec((R, M), lambda i: (i, 0))
        out_shape = jax.ShapeDtypeStruct((N, M), jnp.float32)

    return pl.pallas_call(
        body,
        grid=(grid,),
        in_specs=[
            pl.BlockSpec((R, K), lambda i: (i, 0)),
            pl.BlockSpec((K, M), lambda i: (0, 0)),
            pl.BlockSpec((1, M), lambda i: (0, 0)),
            pl.BlockSpec((1, 1), lambda i: (0, 0)),
            pl.BlockSpec((1, 1), lambda i: (0, 0)),
        ],
        out_specs=out_specs,
        out_shape=out_shape,
    )(x, W, b.reshape(1, M),
      jnp.asarray(a_pre, jnp.float32).reshape(1, 1),
      jnp.asarray(a_post, jnp.float32).reshape(1, 1))


@functools.partial(jax.jit, static_argnames=("post_act",))
def _pl_linear_pair(x0, x1, W, b, a_post, post_act=True):
    N = x0.shape[0]
    M = W.shape[1]
    R = 2048
    grid = _cdiv(N, R)

    def body(x0_ref, x1_ref, w_ref, b_ref, aq_ref, o0_ref, o1_ref):
        y = jnp.dot(x0_ref[...], w_ref[:16, :],
                    preferred_element_type=jnp.float32)
        y = y + jnp.dot(x1_ref[...], w_ref[16:32, :],
                        preferred_element_type=jnp.float32)
        y = y + b_ref[...]
        if post_act:
            aq = aq_ref[0, 0]
            y = jnp.where(y >= 0, y, aq * y)
        o0_ref[...] = y[:, :16]
        o1_ref[...] = y[:, 16:32]

    return pl.pallas_call(
        body,
        grid=(grid,),
        in_specs=[
            pl.BlockSpec((R, 16), lambda i: (i, 0)),
            pl.BlockSpec((R, 16), lambda i: (i, 0)),
            pl.BlockSpec((32, M), lambda i: (0, 0)),
            pl.BlockSpec((1, M), lambda i: (0, 0)),
            pl.BlockSpec((1, 1), lambda i: (0, 0)),
        ],
        out_specs=[pl.BlockSpec((R, 16), lambda i: (i, 0)),
                   pl.BlockSpec((R, 16), lambda i: (i, 0))],
        out_shape=[jax.ShapeDtypeStruct((N, 16), jnp.float32),
                   jax.ShapeDtypeStruct((N, 16), jnp.float32)],
    )(x0, x1, W, b.reshape(1, M),
      jnp.asarray(a_post, jnp.float32).reshape(1, 1))


def _linear(p, x, a=None):
    if a is None:
        return _pl_linear(x, p['W'], p['b'], 0.0, 0.0)
    return _pl_linear(x, p['W'], p['b'], 0.0, a, post_act=True)


def _prelu(x, a):
    return jnp.where(x >= 0, x, a * x)


def _make_sc_prop(E_pad, T, CB):
    CHUNK = CB * 128
    EB = E_pad // NSUB
    NCH = EB // CHUNK
    RPS = EB // 128
    Z = ACC_ROWS // NSUB

    mesh = plsc.VectorSubcoreMesh(core_axis_name="c", subcore_axis_name="s")

    @functools.partial(
        pl.kernel,
        out_type=jax.ShapeDtypeStruct((2, ACC_ROWS, 16), jnp.float32),
        mesh=mesh,
        compiler_params=pltpu.CompilerParams(use_tc_tiling_on_sc=False),
        scratch_types=[
            pltpu.VMEM((T, 16), jnp.float32),
            pltpu.VMEM((2, CB, 128, 16), jnp.float32),
            pltpu.VMEM((2, CB, 4, 128), jnp.int32),
            pltpu.VMEM((16,), jnp.float32),
            pltpu.VMEM_SHARED((ACC_ROWS, 16), jnp.float32),
            pltpu.SemaphoreType.DMA,
            pltpu.SemaphoreType.DMA,
            pltpu.SemaphoreType.DMA,
            pltpu.SemaphoreType.DMA,
        ],
    )
    def sc_prop(u0, u1, g0, g1, idx_h, a_h, zeros_h, out,
                gtab_v, urows_v, si_v, a_v, acc,
                gs0, gs1, ss0, ss1):
        c = lax.axis_index("c")
        s = lax.axis_index("s")
        gsem = (gs0, gs1)
        ssem = (ss0, ss1)

        pltpu.sync_copy(zeros_h.at[pl.ds(s * Z, Z)], acc.at[pl.ds(s * Z, Z)])
        pltpu.sync_copy(a_h, a_v)

        @pl.when(c == 0)
        def _():
            pltpu.sync_copy(g0, gtab_v)

        @pl.when(c == 1)
        def _():
            pltpu.sync_copy(g1, gtab_v)

        plsc.subcore_barrier()

        def fetch_idx(buf, row):
            pltpu.sync_copy(idx_h.at[pl.ds(row, CB)], si_v.at[buf])

        def fire_gathers(buf):
            @pl.when(c == 0)
            def _():
                for b in range(CB):
                    pltpu.async_copy(u0.at[si_v.at[buf, b, 0]],
                                     urows_v.at[buf, b], gsem[buf])

            @pl.when(c == 1)
            def _():
                for b in range(CB):
                    pltpu.async_copy(u1.at[si_v.at[buf, b, 0]],
                                     urows_v.at[buf, b], gsem[buf])

        def drain_gathers(buf):
            for b in range(CB):
                pltpu.make_async_copy(u0.at[si_v.at[buf, b, 0]],
                                      urows_v.at[buf, b], gsem[buf]).wait()

        def fire_scatters(buf):
            for b in range(CB):
                pltpu.async_copy(urows_v.at[buf, b], acc.at[si_v.at[buf, b, 1]],
                                 ssem[buf], add=True)

        def drain_scatters(buf):
            for b in range(CB):
                pltpu.make_async_copy(urows_v.at[buf, b],
                                      acc.at[si_v.at[buf, b, 1]],
                                      ssem[buf]).wait()

        def compute(buf):
            av = a_v[...]
            for b in range(CB):

                def grp(g, carry, _b=b):
                    ts16 = si_v[buf, _b, 2, pl.ds(g * 16, 16)]
                    td16 = si_v[buf, _b, 3, pl.ds(g * 16, 16)]
                    for i in range(16):
                        tsi = ts16[i]
                        tdi = td16[i]
                        z = (urows_v[buf, _b, g * 16 + i]
                             + gtab_v[tsi] - gtab_v[tdi])
                        urows_v[buf, _b, g * 16 + i] = (
                            jnp.maximum(z, 0.0) + av * jnp.minimum(z, 0.0))
                    return carry

                lax.fori_loop(0, 8, grp, 0)

        row0 = s * RPS
        fetch_idx(0, row0)
        fire_gathers(0)

        def step(ch2, carry):
            rowA = row0 + ch2 * 2 * CB
            rowB = rowA + CB
            @pl.when(ch2 >= 1)
            def _():
                drain_scatters(1)

            fetch_idx(1, rowB)
            fire_gathers(1)
            drain_gathers(0)
            compute(0)
            fire_scatters(0)
            drain_scatters(0)

            @pl.when(ch2 <= NCH // 2 - 2)
            def _():
                fetch_idx(0, rowB + CB)
                fire_gathers(0)

            drain_gathers(1)
            compute(1)
            fire_scatters(1)
            return carry

        lax.fori_loop(0, NCH // 2, step, 0)
        drain_scatters(1)
        plsc.subcore_barrier()

        @pl.when(c == 0)
        def _():
            pltpu.sync_copy(acc.at[pl.ds(s * Z, Z)], out.at[0, pl.ds(s * Z, Z)])

        @pl.when(c == 1)
        def _():
            pltpu.sync_copy(acc.at[pl.ds(s * Z, Z)], out.at[1, pl.ds(s * Z, Z)])

    return sc_prop


def _make_sc_count(E_pad):
    RPS = E_pad // NSUB // 128
    Z = ACC_ROWS // NSUB
    mesh = plsc.VectorSubcoreMesh(core_axis_name="c", subcore_axis_name="s")

    @functools.partial(
        pl.kernel,
        out_type=jax.ShapeDtypeStruct((2, ACC_ROWS, 16), jnp.float32),
        mesh=mesh,
        compiler_params=pltpu.CompilerParams(use_tc_tiling_on_sc=False),
        scratch_types=[
            pltpu.VMEM((EB_BLK, 16), jnp.float32),
            pltpu.VMEM((2, 1, EB_BLK), jnp.int32),
            pltpu.VMEM_SHARED((ACC_ROWS, 16), jnp.float32),
            pltpu.SemaphoreType.DMA,
            pltpu.SemaphoreType.DMA,
        ],
    )
    def sc_count(dst_h, ones_h, zeros_h, out, ones_v, di_v, acc, ss0, ss1):
        c = lax.axis_index("c")
        s = lax.axis_index("s")
        ssem = (ss0, ss1)
        pltpu.sync_copy(zeros_h.at[pl.ds(s * Z, Z)], acc.at[pl.ds(s * Z, Z)])
        pltpu.sync_copy(ones_h, ones_v)
        plsc.subcore_barrier()
        row0 = s * RPS

        def blk(j2, carry):
            for buf in range(2):
                @pl.when(j2 >= 1)
                def _(_buf=buf):
                    pltpu.make_async_copy(
                        ones_v, acc.at[di_v.at[_buf, 0]],
                        ssem[_buf]).wait()

                pltpu.sync_copy(dst_h.at[pl.ds(row0 + j2 * 2 + buf, 1)],
                                di_v.at[buf])
                pltpu.async_copy(ones_v, acc.at[di_v.at[buf, 0]],
                                 ssem[buf], add=True)
            return carry

        lax.fori_loop(0, RPS // 2, blk, 0)
        for buf in range(2):
            pltpu.make_async_copy(ones_v, acc.at[di_v.at[buf, 0]],
                                  ssem[buf]).wait()
        plsc.subcore_barrier()

        @pl.when(c == 0)
        def _():
            pltpu.sync_copy(acc.at[pl.ds(s * Z, Z)], out.at[0, pl.ds(s * Z, Z)])

        @pl.when(c == 1)
        def _():
            pltpu.sync_copy(acc.at[pl.ds(s * Z, Z)], out.at[1, pl.ds(s * Z, Z)])

    return sc_count


def _make_sc_edge_mlp(E_pad, T, CB):
    CHUNK = CB * 128
    EB = E_pad // NSUB
    NCH = EB // CHUNK
    RPS = EB // 128
    mesh = plsc.VectorSubcoreMesh(core_axis_name="c", subcore_axis_name="s")

    @functools.partial(
        pl.kernel,
        out_type=jax.ShapeDtypeStruct((2, E_pad // 128, 128, 16), jnp.float32),
        mesh=mesh,
        compiler_params=pltpu.CompilerParams(use_tc_tiling_on_sc=False),
        scratch_types=[
            pltpu.VMEM((T, 16), jnp.float32),
            pltpu.VMEM((2, CB, 128, 16), jnp.float32),
            pltpu.VMEM((2, CB, 128), jnp.int32),
            pltpu.VMEM((2, CB, 128), jnp.int32),
            pltpu.VMEM((2, CB, 128), jnp.int32),
            pltpu.VMEM((16,), jnp.float32),
            pltpu.SemaphoreType.DMA,
            pltpu.SemaphoreType.DMA,
            pltpu.SemaphoreType.DMA,
            pltpu.SemaphoreType.DMA,
        ],
    )
    def sc_emlp(p0, p1, g0, g1, ej_h, ts_h, td_h, a_h, out,
                gtab_v, urows_v, si_v, ts_v, td_v, a_v, gs0, gs1, ws0, ws1):
        c = lax.axis_index("c")
        s = lax.axis_index("s")
        gsem = (gs0, gs1)
        wsem = (ws0, ws1)
        pltpu.sync_copy(a_h, a_v)

        @pl.when(c == 0)
        def _():
            pltpu.sync_copy(g0, gtab_v)

        @pl.when(c == 1)
        def _():
            pltpu.sync_copy(g1, gtab_v)

        def fetch_idx(buf, row):
            pltpu.sync_copy(ej_h.at[pl.ds(row, CB)], si_v.at[buf])
            pltpu.sync_copy(ts_h.at[pl.ds(row, CB)], ts_v.at[buf])
            pltpu.sync_copy(td_h.at[pl.ds(row, CB)], td_v.at[buf])

        def fire_gathers(buf):
            @pl.when(c == 0)
            def _():
                for b in range(CB):
                    pltpu.async_copy(p0.at[si_v.at[buf, b]],
                                     urows_v.at[buf, b], gsem[buf])

            @pl.when(c == 1)
            def _():
                for b in range(CB):
                    pltpu.async_copy(p1.at[si_v.at[buf, b]],
                                     urows_v.at[buf, b], gsem[buf])

        def drain_gathers(buf):
            for b in range(CB):
                pltpu.make_async_copy(p0.at[si_v.at[buf, b]],
                                      urows_v.at[buf, b], gsem[buf]).wait()

        def fire_writes(buf, row):
            @pl.when(c == 0)
            def _():
                pltpu.async_copy(urows_v.at[buf], out.at[0, pl.ds(row, CB)],
                                 wsem[buf])

            @pl.when(c == 1)
            def _():
                pltpu.async_copy(urows_v.at[buf], out.at[1, pl.ds(row, CB)],
                                 wsem[buf])

        def drain_writes(buf, row):
            pltpu.make_async_copy(urows_v.at[buf], out.at[0, pl.ds(row, CB)],
                                  wsem[buf]).wait()

        def compute(buf):
            av = a_v[...]
            for b in range(CB):

                def grp(g, carry, _b=b):
                    ts16 = ts_v[buf, _b, pl.ds(g * 16, 16)]
                    td16 = td_v[buf, _b, pl.ds(g * 16, 16)]
                    for i in range(16):
                        tsi = ts16[i]
                        tdi = td16[i]
                        z = (urows_v[buf, _b, g * 16 + i]
                             + gtab_v[tsi] - gtab_v[tdi])
                        urows_v[buf, _b, g * 16 + i] = (
                            jnp.maximum(z, 0.0) + av * jnp.minimum(z, 0.0))
                    return carry

                lax.fori_loop(0, 8, grp, 0)

        row0 = s * RPS
        fetch_idx(0, row0)
        fire_gathers(0)

        def step(ch2, carry):
            rowA = row0 + ch2 * 2 * CB
            rowB = rowA + CB

            @pl.when(ch2 >= 1)
            def _():
                drain_writes(1, rowB - 2 * CB)

            fetch_idx(1, rowB)
            fire_gathers(1)
            drain_gathers(0)
            compute(0)
            fire_writes(0, rowA)
            drain_writes(0, rowA)

            @pl.when(ch2 <= NCH // 2 - 2)
            def _():
                fetch_idx(0, rowB + CB)
                fire_gathers(0)

            drain_gathers(1)
            compute(1)
            fire_writes(1, rowB)
            return carry

        lax.fori_loop(0, NCH // 2, step, 0)
        drain_writes(1, row0 + (NCH - 1) * CB)

    return sc_emlp


SEG_ROWS = 1024
SEG_JUNK = 1000


def _make_sc_seg_sum(E_pad):
    RPS = E_pad // NSUB // 128
    Z = SEG_ROWS // NSUB
    mesh = plsc.VectorSubcoreMesh(core_axis_name="c", subcore_axis_name="s")

    @functools.partial(
        pl.kernel,
        out_type=jax.ShapeDtypeStruct((2, SEG_ROWS, 16), jnp.float32),
        mesh=mesh,
        compiler_params=pltpu.CompilerParams(use_tc_tiling_on_sc=False),
        scratch_types=[
            pltpu.VMEM((2, 1, 128, 16), jnp.float32),
            pltpu.VMEM((2, 1, 128), jnp.int32),
            pltpu.VMEM_SHARED((SEG_ROWS, 16), jnp.float32),
            pltpu.SemaphoreType.DMA,
            pltpu.SemaphoreType.DMA,
        ],
    )
    def sc_seg(h0, h1, ei_h, zeros_h, out, pay_v, di_v, acc, ss0, ss1):
        c = lax.axis_index("c")
        s = lax.axis_index("s")
        ssem = (ss0, ss1)
        pltpu.sync_copy(zeros_h.at[pl.ds(s * Z, Z)], acc.at[pl.ds(s * Z, Z)])
        plsc.subcore_barrier()
        row0 = s * RPS

        def blk(j2, carry):
            for buf in range(2):
                @pl.when(j2 >= 1)
                def _(_buf=buf):
                    pltpu.make_async_copy(pay_v.at[_buf, 0],
                                          acc.at[di_v.at[_buf, 0]],
                                          ssem[_buf]).wait()

                row = row0 + j2 * 2 + buf
                pltpu.sync_copy(ei_h.at[pl.ds(row, 1)], di_v.at[buf])

                @pl.when(c == 0)
                def _(_buf=buf, _row=row):
                    pltpu.sync_copy(h0.at[pl.ds(_row, 1)], pay_v.at[_buf])

                @pl.when(c == 1)
                def _(_buf=buf, _row=row):
                    pltpu.sync_copy(h1.at[pl.ds(_row, 1)], pay_v.at[_buf])

                pltpu.async_copy(pay_v.at[buf, 0], acc.at[di_v.at[buf, 0]],
                                 ssem[buf], add=True)
            return carry

        lax.fori_loop(0, RPS // 2, blk, 0)
        for buf in range(2):
            pltpu.make_async_copy(pay_v.at[buf, 0], acc.at[di_v.at[buf, 0]],
                                  ssem[buf]).wait()
        plsc.subcore_barrier()

        @pl.when(c == 0)
        def _():
            pltpu.sync_copy(acc.at[pl.ds(s * Z, Z)], out.at[0, pl.ds(s * Z, Z)])

        @pl.when(c == 1)
        def _():
            pltpu.sync_copy(acc.at[pl.ds(s * Z, Z)], out.at[1, pl.ds(s * Z, Z)])

    return sc_seg


def _pad_edges(v, E_pad, fill):
    E = v.shape[0]
    return jnp.pad(v, (0, E_pad - E), constant_values=fill)


def kernel(x, mask, A_in_events, A_in_clusters, A_src_in_product, A_src_in_sta,
           locs_cart, srcs_cart, params):
    n = x.shape[0]
    rel = locs_cart[A_src_in_sta[0]] - srcs_cart[A_src_in_sta[1]]
    x = jnp.concatenate([x, rel], axis=1)
    mask = jnp.concatenate([mask, rel], axis=1)
    e = params['embed']
    mask = _linear(e['l2'], _linear(e['l1'], mask, e['a']))
    sta_of_node = A_src_in_sta[0].astype(jnp.int32)
    src_of_node = A_src_in_sta[1].astype(jnp.int32)

    E = A_in_events.shape[1]
    E_pad = _cdiv(E, NSUB * 1024) * NSUB * 1024
    sc_prop_ev = _make_sc_prop(E_pad, 512, 4)
    sc_prop_cl = _make_sc_prop(E_pad, 1024, 2)
    sc_count = _make_sc_count(E_pad)

    zeros_h = jnp.zeros((ACC_ROWS, 16), jnp.float32)
    ones_h = jnp.ones((EB_BLK, 16), jnp.float32)

    def prep_edges(A, tbl_of_node):
        src = _pad_edges(A[0].astype(jnp.int32), E_pad, 0).reshape(-1, 128)
        dst = _pad_edges(A[1].astype(jnp.int32), E_pad, JUNK_ROW).reshape(-1, 128)
        ts = _pad_edges(tbl_of_node[A[0]], E_pad, 0).reshape(-1, 128)
        td = _pad_edges(tbl_of_node[A[1]], E_pad, 0).reshape(-1, 128)
        idx4 = jnp.stack([src, dst, ts, td], axis=1)
        cnt = sc_count(dst, ones_h, zeros_h)[0][:N_NODES, :1]
        recip = 1.0 / jnp.maximum(cnt, 1.0)
        return idx4, recip

    ev_edges, ev_recip = prep_edges(A_in_events, sta_of_node)
    cl_edges, cl_recip = prep_edges(A_in_clusters, src_of_node)

    locs_pad = jnp.pad(locs_cart, ((0, 512 - locs_cart.shape[0]), (0, 0)))
    srcs_pad = jnp.pad(srcs_cart, ((0, 1024 - srcs_cart.shape[0]), (0, 0)))

    def prop(sc_prop, p, edges, recip, base_tab, xin_pre, a_in):
        Wm, bm = p['merge']['W'], p['merge']['b']
        W_node = jnp.pad(Wm[:30], ((0, 0), (0, 2)))
        b_node = jnp.pad(bm, (0, 2))
        u0, u1 = _pl_linear(xin_pre, W_node, b_node, a_in, 0.0,
                            pre_act=True, split16=True)
        gt = base_tab @ jnp.pad(Wm[30:34], ((0, 0), (0, 2)))
        gt0, gt1 = gt[:, :16], gt[:, 16:32]
        a_vec = jnp.full((16,), p['a_merge'], jnp.float32)
        out = sc_prop(u0, u1, gt0, gt1, edges, a_vec, zeros_h)
        o = jnp.concatenate([out[0][:N_NODES], out[1][:N_NODES]], axis=1)[:, :30]
        return o * recip

    def data_agg(p, xx):
        tr = jnp.concatenate([xx, mask], axis=1)
        tr = _linear(p['init_trns'], tr, p['a_init'])
        o1 = prop(sc_prop_ev, p, ev_edges, ev_recip, locs_pad, tr, p['a11'])
        o2 = prop(sc_prop_cl, p, cl_edges, cl_recip, srcs_pad, tr, p['a12'])
        tr1 = _linear(p['l1_t1_2'], jnp.concatenate([tr, o1, mask], axis=1))
        tr2 = _linear(p['l1_t2_2'], jnp.concatenate([tr, o2, mask], axis=1))
        tr = _prelu(jnp.concatenate([tr1, tr2], axis=1), p['a1'])
        h1 = _linear(p['l2_t1_1'], tr, p['a21'])
        h2 = _linear(p['l2_t2_1'], tr, p['a22'])
        o1 = prop(sc_prop_ev, p, ev_edges, ev_recip, locs_pad, h1, jnp.float32(1.0))
        o2 = prop(sc_prop_cl, p, cl_edges, cl_recip, srcs_pad, h2, jnp.float32(1.0))
        tr1 = _linear(p['l2_t1_2'], jnp.concatenate([tr, o1, mask], axis=1))
        tr2 = _linear(p['l2_t2_2'], jnp.concatenate([tr, o2, mask], axis=1))
        return _prelu(jnp.concatenate([tr1, tr2], axis=1), p['a2'])

    for p in params['aggs']:
        x = data_agg(p, x)

    b = params['bip']
    xcat = jnp.concatenate([x, mask], axis=1)
    ej = A_src_in_product[0].astype(jnp.int32)
    ei = A_src_in_product[1].astype(jnp.int32)
    sta_ej = sta_of_node[ej]
    W1 = b['fc1a']['W']
    W1x = jnp.pad(W1[:40], ((0, 0), (0, 2)))
    W1p = jnp.pad(W1[40:44], ((0, 0), (0, 2)))
    b1 = jnp.pad(b['fc1a']['b'], (0, 2))
    P00, P01 = _pl_linear(xcat, W1x, b1, 0.0, 0.0, split16=True)
    lt = jnp.pad(locs_cart @ W1p, ((0, 12), (0, 0)))
    Q = jnp.pad(srcs_cart @ W1p, ((0, 24), (0, 0)))
    bigtab = jnp.concatenate([-lt, -Q], axis=0)

    E2 = ej.shape[0]
    E_pad2 = _cdiv(E2, NSUB * 1024) * NSUB * 1024
    sc_emlp = _make_sc_edge_mlp(E_pad2, 1536, 4)
    sc_seg = _make_sc_seg_sum(E_pad2)

    ej_p = _pad_edges(ej, E_pad2, 0).reshape(-1, 128)
    ts_p = _pad_edges(sta_ej, E_pad2, 0).reshape(-1, 128)
    td_p = _pad_edges(512 + ei, E_pad2, 512 + SEG_JUNK).reshape(-1, 128)
    ei_p = _pad_edges(ei, E_pad2, SEG_JUNK).reshape(-1, 128)
    a_fc1_vec = jnp.full((16,), b['a_fc1'], jnp.float32)
    Y = sc_emlp(P00, P01, bigtab[:, :16], bigtab[:, 16:32],
                ej_p, ts_p, td_p, a_fc1_vec)
    Y0 = Y[0].reshape(E_pad2, 16)
    Y1 = Y[1].reshape(E_pad2, 16)
    W2 = jnp.pad(b['fc1b']['W'], ((0, 2), (0, 2)))
    b2 = jnp.pad(b['fc1b']['b'], (0, 2))
    H0, H1 = _pl_linear_pair(Y0, Y1, W2, b2, b['a1'])
    seg = sc_seg(H0.reshape(-1, 128, 16), H1.reshape(-1, 128, 16),
                 ei_p, zeros_h[:SEG_ROWS])
    nsrc = srcs_cart.shape[0]
    agg = jnp.concatenate([seg[0][:nsrc], seg[1][:nsrc]], axis=1)[:, :30]
    out = _linear(b['fc2'], agg, b['a2'])
    pr = params['proj']
    return _linear(pr['l2'], _linear(pr['l1'], out, pr['a']))
